# baseline (speedup 1.0000x reference)
import functools

import jax
import jax.numpy as jnp
from jax import lax
from jax.experimental import pallas as pl
from jax.experimental.pallas import tpu as pltpu
from jax.experimental.pallas import tpu_sc as plsc

_BATCH = 4096
_HIST = 200
_EMBED_DIM = 64


@functools.lru_cache(maxsize=None)
def _build_gather(total_rows: int, dim: int):
  info = plsc.get_sparse_core_info()
  nc, ns = info.num_cores, info.num_subcores
  nw = nc * ns
  assert total_rows % nw == 0
  rows_per_w = total_rows // nw
  chunk = 512
  assert rows_per_w % chunk == 0
  n_chunks = rows_per_w // chunk

  mesh = plsc.VectorSubcoreMesh(core_axis_name="c", subcore_axis_name="s")

  @functools.partial(
      pl.kernel,
      mesh=mesh,
      out_type=jax.ShapeDtypeStruct((total_rows, dim), jnp.float32),
      scratch_types=[
          pltpu.VMEM((rows_per_w,), jnp.int32),
          pltpu.VMEM((chunk, dim), jnp.float32),
          pltpu.SemaphoreType.DMA,
      ],
      compiler_params=pltpu.CompilerParams(use_tc_tiling_on_sc=False),
  )
  def gather_kernel(idx_hbm, table_hbm, out_hbm, idx_v, rows_v, sem):
    wid = lax.axis_index("s") * nc + lax.axis_index("c")
    base = wid * rows_per_w
    pltpu.sync_copy(idx_hbm.at[pl.ds(base, rows_per_w)], idx_v)

    def body(g, carry):
      off = g * chunk
      pltpu.async_copy(
          table_hbm.at[idx_v.at[pl.ds(off, chunk)]], rows_v, sem
      ).wait()
      pltpu.sync_copy(rows_v, out_hbm.at[pl.ds(base + off, chunk)])
      return carry

    lax.fori_loop(0, n_chunks, body, 0)

  return gather_kernel


def kernel(x, table):
  flat_idx = x.reshape(-1).astype(jnp.int32)
  out = _build_gather(flat_idx.shape[0], table.shape[1])(flat_idx, table)
  return out.reshape(x.shape[0], x.shape[1], table.shape[1])


# speedup vs baseline: 1.0200x; 1.0200x over previous
import functools

import jax
import jax.numpy as jnp
from jax import lax
from jax.experimental import pallas as pl
from jax.experimental.pallas import tpu as pltpu
from jax.experimental.pallas import tpu_sc as plsc

_NBUF = 4
_CHUNK = 400


@functools.lru_cache(maxsize=None)
def _build_gather(total_rows: int, dim: int):
  info = plsc.get_sparse_core_info()
  nc, ns = info.num_cores, info.num_subcores
  nw = nc * ns
  assert total_rows % nw == 0
  rows_per_w = total_rows // nw
  chunk = _CHUNK
  nbuf = _NBUF
  assert rows_per_w % chunk == 0
  n_chunks = rows_per_w // chunk
  assert n_chunks % nbuf == 0
  n_rounds = n_chunks // nbuf

  mesh = plsc.VectorSubcoreMesh(core_axis_name="c", subcore_axis_name="s")

  @functools.partial(
      pl.kernel,
      mesh=mesh,
      out_type=jax.ShapeDtypeStruct((total_rows, dim), jnp.float32),
      scratch_types=[
          pltpu.VMEM((rows_per_w,), jnp.int32),
          pltpu.VMEM((nbuf, chunk, dim), jnp.float32),
          pltpu.SemaphoreType.DMA((nbuf,)),
          pltpu.SemaphoreType.DMA((nbuf,)),
      ],
      compiler_params=pltpu.CompilerParams(use_tc_tiling_on_sc=False),
  )
  def gather_kernel(idx_hbm, table_hbm, out_hbm, idx_v, rows_v, gsem, wsem):
    wid = lax.axis_index("s") * nc + lax.axis_index("c")
    base = wid * rows_per_w
    pltpu.sync_copy(idx_hbm.at[pl.ds(base, rows_per_w)], idx_v)

    def gather(t, b):
      pltpu.async_copy(
          table_hbm.at[idx_v.at[pl.ds(t * chunk, chunk)]],
          rows_v.at[b],
          gsem.at[b],
      )

    def wait_gather(t, b):
      pltpu.make_async_copy(
          table_hbm.at[idx_v.at[pl.ds(t * chunk, chunk)]],
          rows_v.at[b],
          gsem.at[b],
      ).wait()

    for b in range(nbuf):
      gather(b, b)

    def body(r, carry):
      for b in range(nbuf):
        t = r * nbuf + b
        wait_gather(t, b)
        pltpu.async_copy(
            rows_v.at[b],
            out_hbm.at[pl.ds(base + t * chunk, chunk)],
            wsem.at[b],
        ).wait()

        @pl.when(t + nbuf < n_chunks)
        def _():
          gather(t + nbuf, b)

      return carry

    lax.fori_loop(0, n_rounds, body, 0)

  return gather_kernel


def kernel(x, table):
  flat_idx = x.reshape(-1).astype(jnp.int32)
  out = _build_gather(flat_idx.shape[0], table.shape[1])(flat_idx, table)
  return out.reshape(x.shape[0], x.shape[1], table.shape[1])


# speedup vs baseline: 1.0864x; 1.0651x over previous
import functools

import jax
import jax.numpy as jnp
from jax import lax
from jax.experimental import pallas as pl
from jax.experimental.pallas import tpu as pltpu
from jax.experimental.pallas import tpu_sc as plsc

_NBUF = 4
_CHUNK = 200


@functools.lru_cache(maxsize=None)
def _build_gather(total_rows: int, vocab: int, wdim: int):
  info = plsc.get_sparse_core_info()
  nc, ns = info.num_cores, info.num_subcores
  nw = nc * ns
  assert total_rows % nw == 0
  rows_per_w = total_rows // nw
  chunk = _CHUNK
  nbuf = _NBUF
  assert rows_per_w % chunk == 0
  n_chunks = rows_per_w // chunk
  assert n_chunks % nbuf == 0
  n_rounds = n_chunks // nbuf

  mesh = plsc.VectorSubcoreMesh(core_axis_name="c", subcore_axis_name="s")

  @functools.partial(
      pl.kernel,
      mesh=mesh,
      out_type=jax.ShapeDtypeStruct((total_rows, wdim), jnp.float32),
      scratch_types=[
          pltpu.VMEM((rows_per_w,), jnp.int32),
          pltpu.VMEM((nbuf, chunk, wdim), jnp.float32),
          pltpu.SemaphoreType.DMA((nbuf,)),
          pltpu.SemaphoreType.DMA((nbuf,)),
      ],
      compiler_params=pltpu.CompilerParams(use_tc_tiling_on_sc=False),
  )
  def gather_kernel(idx_hbm, table_hbm, out_hbm, idx_v, rows_v, gsem, wsem):
    wid = lax.axis_index("s") * nc + lax.axis_index("c")
    base = wid * rows_per_w
    pltpu.sync_copy(idx_hbm.at[pl.ds(base, rows_per_w)], idx_v)

    def gather(t, b):
      pltpu.async_copy(
          table_hbm.at[idx_v.at[pl.ds(t * chunk, chunk)]],
          rows_v.at[b],
          gsem.at[b],
      )

    def wait_gather(t, b):
      pltpu.make_async_copy(
          table_hbm.at[idx_v.at[pl.ds(t * chunk, chunk)]],
          rows_v.at[b],
          gsem.at[b],
      ).wait()

    for b in range(nbuf):
      gather(b, b)

    def body(r, carry):
      for b in range(nbuf):
        t = r * nbuf + b
        wait_gather(t, b)
        pltpu.async_copy(
            rows_v.at[b],
            out_hbm.at[pl.ds(base + t * chunk, chunk)],
            wsem.at[b],
        ).wait()

        @pl.when(t + nbuf < n_chunks)
        def _():
          gather(t + nbuf, b)

      return carry

    lax.fori_loop(0, n_rounds, body, 0)

  return gather_kernel


def kernel(x, table):
  flat_idx = x.reshape(-1).astype(jnp.int32)
  vocab, dim = table.shape
  table_wide = jnp.concatenate([table, table], axis=1)
  out = _build_gather(flat_idx.shape[0], vocab, 2 * dim)(flat_idx, table_wide)
  return out[:, :dim].reshape(x.shape[0], x.shape[1], dim)


# speedup vs baseline: 1.3597x; 1.2516x over previous
import functools

import jax
import jax.numpy as jnp
from jax import lax
from jax.experimental import pallas as pl
from jax.experimental.pallas import tpu as pltpu
from jax.experimental.pallas import tpu_sc as plsc

_NBUF = 4
_CHUNK = 400


@functools.lru_cache(maxsize=None)
def _build_gather(total_rows: int, vocab: int, dim: int):
  info = plsc.get_sparse_core_info()
  nc, ns = info.num_cores, info.num_subcores
  nw = nc * ns
  assert total_rows % nw == 0
  rows_per_w = total_rows // nw
  chunk = _CHUNK
  nbuf = _NBUF
  assert rows_per_w % chunk == 0
  n_chunks = rows_per_w // chunk
  assert n_chunks % nbuf == 0
  n_rounds = n_chunks // nbuf

  mesh = plsc.VectorSubcoreMesh(core_axis_name="c", subcore_axis_name="s")

  @functools.partial(
      pl.kernel,
      mesh=mesh,
      out_type=jax.ShapeDtypeStruct((total_rows, 2 * dim), jnp.float32),
      scratch_types=[
          pltpu.VMEM((rows_per_w,), jnp.int32),
          pltpu.VMEM((nbuf, chunk, dim), jnp.float32),
          pltpu.SemaphoreType.DMA((nbuf,)),
          pltpu.SemaphoreType.DMA((nbuf,)),
      ],
      compiler_params=pltpu.CompilerParams(use_tc_tiling_on_sc=False),
  )
  def gather_kernel(idx_hbm, table_hbm, out_hbm, idx_v, rows_v, gsem, wsem):
    wid = lax.axis_index("s") * nc + lax.axis_index("c")
    base = wid * rows_per_w
    pltpu.sync_copy(idx_hbm.at[pl.ds(base, rows_per_w)], idx_v)

    def gather(t, b):
      pltpu.async_copy(
          table_hbm.at[idx_v.at[pl.ds(t * chunk, chunk)]],
          rows_v.at[b],
          gsem.at[b],
      )

    def wait_gather(t, b):
      pltpu.make_async_copy(
          table_hbm.at[idx_v.at[pl.ds(t * chunk, chunk)]],
          rows_v.at[b],
          gsem.at[b],
      ).wait()

    for b in range(nbuf):
      gather(b, b)

    def body(r, carry):
      for b in range(nbuf):
        t = r * nbuf + b
        wait_gather(t, b)
        pltpu.async_copy(
            rows_v.at[b],
            out_hbm.at[pl.ds(base + t * chunk, chunk), pl.ds(0, dim)],
            wsem.at[b],
        ).wait()

        @pl.when(t + nbuf < n_chunks)
        def _():
          gather(t + nbuf, b)

      return carry

    lax.fori_loop(0, n_rounds, body, 0)

  return gather_kernel


def kernel(x, table):
  flat_idx = x.reshape(-1).astype(jnp.int32)
  vocab, dim = table.shape
  table_lin = lax.optimization_barrier(table.reshape(-1))
  table2d = table_lin.reshape(vocab, dim)
  out = _build_gather(flat_idx.shape[0], vocab, dim)(flat_idx, table2d)
  return out[:, :dim].reshape(x.shape[0], x.shape[1], dim)


# speedup vs baseline: 1.3614x; 1.0013x over previous
import functools

import jax
import jax.numpy as jnp
from jax import lax
from jax.experimental import pallas as pl
from jax.experimental.pallas import tpu as pltpu
from jax.experimental.pallas import tpu_sc as plsc

_NBUF = 4
_CHUNK = 400


@functools.lru_cache(maxsize=None)
def _build_gather(total_rows: int, vocab: int, dim: int):
  info = plsc.get_sparse_core_info()
  nc, ns = info.num_cores, info.num_subcores
  nw = nc * ns
  assert total_rows % nw == 0
  rows_per_w = total_rows // nw
  chunk = _CHUNK
  nbuf = _NBUF
  assert rows_per_w % chunk == 0
  n_chunks = rows_per_w // chunk
  assert n_chunks % nbuf == 0
  n_rounds = n_chunks // nbuf

  mesh = plsc.VectorSubcoreMesh(core_axis_name="c", subcore_axis_name="s")

  @functools.partial(
      pl.kernel,
      mesh=mesh,
      out_type=jax.ShapeDtypeStruct((total_rows, 2 * dim), jnp.float32),
      scratch_types=[
          pltpu.VMEM((rows_per_w,), jnp.int32),
          pltpu.VMEM((nbuf, chunk, dim), jnp.float32),
          pltpu.SemaphoreType.DMA((nbuf,)),
          pltpu.SemaphoreType.DMA((nbuf,)),
      ],
      compiler_params=pltpu.CompilerParams(use_tc_tiling_on_sc=False),
  )
  def gather_kernel(idx_hbm, table_hbm, out_hbm, idx_v, rows_v, gsem, wsem):
    wid = lax.axis_index("s") * nc + lax.axis_index("c")
    base = wid * rows_per_w
    head = nbuf * chunk
    pltpu.sync_copy(idx_hbm.at[pl.ds(base, head)], idx_v.at[pl.ds(0, head)])

    def gather(t, b):
      pltpu.async_copy(
          table_hbm.at[idx_v.at[pl.ds(t * chunk, chunk)]],
          rows_v.at[b],
          gsem.at[b],
      )

    def wait_gather(t, b):
      pltpu.make_async_copy(
          table_hbm.at[idx_v.at[pl.ds(t * chunk, chunk)]],
          rows_v.at[b],
          gsem.at[b],
      ).wait()

    for b in range(nbuf):
      gather(b, b)
    pltpu.sync_copy(
        idx_hbm.at[pl.ds(base + head, rows_per_w - head)],
        idx_v.at[pl.ds(head, rows_per_w - head)],
    )

    def body(r, carry):
      for b in range(nbuf):
        t = r * nbuf + b
        wait_gather(t, b)
        pltpu.async_copy(
            rows_v.at[b],
            out_hbm.at[pl.ds(base + t * chunk, chunk), pl.ds(0, dim)],
            wsem.at[b],
        ).wait()

        @pl.when(t + nbuf < n_chunks)
        def _():
          gather(t + nbuf, b)

      return carry

    lax.fori_loop(0, n_rounds, body, 0)

  return gather_kernel


def kernel(x, table):
  flat_idx = x.reshape(-1).astype(jnp.int32)
  vocab, dim = table.shape
  table_lin = lax.optimization_barrier(table.reshape(-1))
  table2d = table_lin.reshape(vocab, dim)
  out = _build_gather(flat_idx.shape[0], vocab, dim)(flat_idx, table2d)
  return out[:, :dim].reshape(x.shape[0], x.shape[1], dim)
